# baseline (speedup 1.0000x reference)
import functools

import jax
import jax.numpy as jnp
from jax import lax
from jax.experimental import pallas as pl
from jax.experimental.pallas import tpu as pltpu
from jax.experimental.pallas import tpu_sc as plsc

N_NODES = 10000
N_PAD = 10240
N_EDGES = 320000
NFEAT = 128
NHID = 32
NCLASS = 32
KAPPA = 0.9
FP_ITERS = 15
N_LAYERS = 5

NC = 2
NS = 16
CHUNK = 128
NCHUNK = 162
EPT = NCHUNK * CHUNK
E_PAD = NS * EPT
RING = 10
LAG_S = 4
LAG_I = 3
ROWS_PER_TILE = N_PAD // NS


def _spmm_body(mc, x_hbm, idx_hbm, zeros_hbm, out_hbm,
               idxr_v, rows_v, acc_sh, isem, gsem, ssem):
    cid = lax.axis_index("c")
    sid = lax.axis_index("s")

    pltpu.sync_copy(zeros_hbm.at[pl.ds(sid * ROWS_PER_TILE, ROWS_PER_TILE)],
                    acc_sh.at[pl.ds(sid * ROWS_PER_TILE, ROWS_PER_TILE)])
    plsc.subcore_barrier()

    for j in range(LAG_I):
        pltpu.async_copy(idx_hbm.at[cid, sid, j], idxr_v.at[j], isem)

    def step(k, c):
        ki = k + LAG_I

        @pl.when(ki < NCHUNK)
        def _():
            pltpu.async_copy(idx_hbm.at[cid, sid, ki],
                             idxr_v.at[lax.rem(ki, RING)], isem)

        @pl.when(k < NCHUNK)
        def _():
            b = lax.rem(k, RING)
            pltpu.make_async_copy(idx_hbm.at[cid, sid, k], idxr_v.at[b],
                                  isem).wait()

            @pl.when(k >= RING)
            def _():
                bb = lax.rem(k - RING, RING)
                pltpu.make_async_copy(rows_v.at[bb],
                                      acc_sh.at[idxr_v.at[bb, 1]],
                                      ssem).wait()

            pltpu.async_copy(x_hbm.at[idxr_v.at[b, 0]], rows_v.at[b], gsem)

        @pl.when(k >= LAG_S)
        def _():
            b2 = lax.rem(k - LAG_S, RING)
            pltpu.make_async_copy(x_hbm.at[idxr_v.at[b2, 0]], rows_v.at[b2],
                                  gsem).wait()
            pltpu.async_copy(rows_v.at[b2], acc_sh.at[idxr_v.at[b2, 1]],
                             ssem, add=True)
        return c

    lax.fori_loop(0, NCHUNK + LAG_S, step, 0)

    def drain(k, c):
        b = lax.rem(k, RING)
        pltpu.make_async_copy(rows_v.at[b], acc_sh.at[idxr_v.at[b, 1]],
                              ssem).wait()
        return c
    lax.fori_loop(NCHUNK - RING, NCHUNK, drain, 0)
    plsc.subcore_barrier()

    pltpu.sync_copy(acc_sh.at[pl.ds(sid * ROWS_PER_TILE, ROWS_PER_TILE)],
                    out_hbm.at[cid, pl.ds(sid * ROWS_PER_TILE, ROWS_PER_TILE)])


@functools.lru_cache(maxsize=None)
def _make_spmm(mc):
    mesh = plsc.VectorSubcoreMesh(core_axis_name="c", subcore_axis_name="s",
                                  num_cores=NC, num_subcores=NS)
    return pl.kernel(
        functools.partial(_spmm_body, mc),
        out_type=jax.ShapeDtypeStruct((NC, N_PAD, mc), jnp.float32),
        mesh=mesh,
        compiler_params=pltpu.CompilerParams(use_tc_tiling_on_sc=False),
        scratch_types=[
            pltpu.VMEM((RING, 2, CHUNK), jnp.int32),
            pltpu.VMEM((RING, CHUNK, mc), jnp.float32),
            pltpu.VMEM_SHARED((N_PAD, mc), jnp.float32),
            pltpu.SemaphoreType.DMA,
            pltpu.SemaphoreType.DMA,
            pltpu.SemaphoreType.DMA,
        ],
    )


def _spmm(x_stk, idx5):
    mc = x_stk.shape[2]
    zeros = jnp.zeros((N_PAD, mc), jnp.float32)
    return _make_spmm(mc)(x_stk.reshape(2 * N_PAD, mc), idx5, zeros)


def _stk_matmul(s_stk, Wt):
    m_in = 2 * s_stk.shape[2]
    m_out = Wt.shape[1]
    W4 = Wt.reshape(2, m_in // 2, 2, m_out // 2)
    return jnp.einsum("cnk,ckdj->dnj", s_stk, W4)


SP_CHUNK = 128
SP_NCHUNK = 162
SP_ITERS = 50


def _spectral_body(src_hbm, dst_hbm, vinit_hbm, outa_hbm, outb_hbm,
                   sidx_v, didx_v, av_v, vals_v, vsl_v, zb_v, obuf_v, sums_v,
                   v_sh, av_sh, gsem, ssem):
    cid = lax.axis_index("c")
    sid = lax.axis_index("s")
    rsl = pl.ds(sid * ROWS_PER_TILE, ROWS_PER_TILE)

    pltpu.sync_copy(src_hbm.at[sid], sidx_v)
    pltpu.sync_copy(dst_hbm.at[sid], didx_v)
    pltpu.sync_copy(vinit_hbm.at[rsl], v_sh.at[rsl])

    def z_body(i, c):
        zb_v[pl.ds(pl.multiple_of(i * 16, 16), 16)] = jnp.zeros((16,), jnp.float32)
        return c
    lax.fori_loop(0, ROWS_PER_TILE // 16, z_body, 0)
    pltpu.sync_copy(zb_v, av_sh.at[rsl])
    plsc.subcore_barrier()

    def iter_body(t, c0):
        def g_body(k, c):
            pltpu.async_copy(v_sh.at[sidx_v.at[k]], vals_v.at[k], gsem)
            return c
        lax.fori_loop(0, SP_NCHUNK, g_body, 0)

        def gw_body(k, c):
            pltpu.make_async_copy(v_sh.at[sidx_v.at[k]], vals_v.at[k],
                                  gsem).wait()
            return c
        lax.fori_loop(0, SP_NCHUNK, gw_body, 0)

        def s_body(k, c):
            pltpu.async_copy(vals_v.at[k], av_sh.at[didx_v.at[k]],
                             ssem, add=True)
            return c
        lax.fori_loop(0, SP_NCHUNK, s_body, 0)

        def w_body(k, c):
            pltpu.make_async_copy(vals_v.at[k], av_sh.at[didx_v.at[k]],
                                  ssem).wait()
            return c
        lax.fori_loop(0, SP_NCHUNK, w_body, 0)
        plsc.subcore_barrier()

        pltpu.sync_copy(av_sh, av_v)
        plsc.subcore_barrier()
        pltpu.sync_copy(zb_v, av_sh.at[rsl])

        obuf_v[...] = jnp.zeros((16,), jnp.float32)
        def r_body(i, c):
            xv = av_v[pl.ds(pl.multiple_of(i * 16, 16), 16)]
            obuf_v[...] = obuf_v[...] + xv * xv
            return c
        lax.fori_loop(0, N_NODES // 16, r_body, 0)
        sums_v[pl.ds(pl.multiple_of((t % 2) * 16, 16), 16)] = obuf_v[...]

        def v_body(i, c):
            vsl_v[pl.ds(pl.multiple_of(i * 16, 16), 16)] = (
                av_v[pl.ds(pl.multiple_of(sid * ROWS_PER_TILE + i * 16, 16), 16)]
                * jnp.float32(1.0 / 32.0))
            return c
        lax.fori_loop(0, ROWS_PER_TILE // 16, v_body, 0)
        pltpu.sync_copy(vsl_v, v_sh.at[rsl])
        plsc.subcore_barrier()
        return c0

    lax.fori_loop(0, SP_ITERS, iter_body, 0)

    @pl.when(jnp.logical_and(cid == 0, sid == 0))
    def _():
        obuf_v[...] = sums_v[pl.ds(16, 16)]
        pltpu.sync_copy(obuf_v, outa_hbm)
        obuf_v[...] = sums_v[pl.ds(0, 16)]
        pltpu.sync_copy(obuf_v, outb_hbm)


@functools.lru_cache(maxsize=None)
def _make_spectral():
    mesh = plsc.VectorSubcoreMesh(core_axis_name="c", subcore_axis_name="s",
                                  num_cores=NC, num_subcores=NS)
    return pl.kernel(
        _spectral_body,
        out_type=[jax.ShapeDtypeStruct((16,), jnp.float32),
                  jax.ShapeDtypeStruct((16,), jnp.float32)],
        mesh=mesh,
        compiler_params=pltpu.CompilerParams(use_tc_tiling_on_sc=False),
        scratch_types=[
            pltpu.VMEM((SP_NCHUNK, SP_CHUNK), jnp.int32),
            pltpu.VMEM((SP_NCHUNK, SP_CHUNK), jnp.int32),
            pltpu.VMEM((N_PAD,), jnp.float32),
            pltpu.VMEM((SP_NCHUNK, SP_CHUNK), jnp.float32),
            pltpu.VMEM((ROWS_PER_TILE,), jnp.float32),
            pltpu.VMEM((ROWS_PER_TILE,), jnp.float32),
            pltpu.VMEM((16,), jnp.float32),
            pltpu.VMEM((32,), jnp.float32),
            pltpu.VMEM_SHARED((N_PAD,), jnp.float32),
            pltpu.VMEM_SHARED((N_PAD,), jnp.float32),
            pltpu.SemaphoreType.DMA,
            pltpu.SemaphoreType.DMA,
        ],
    )


def _proj_linf(W, v):
    a_abs = jnp.abs(W)
    ssort = -jnp.sort(-a_abs, axis=1)
    cssv = jnp.cumsum(ssort, axis=1) - v
    ind = jnp.arange(1, W.shape[1] + 1, dtype=W.dtype)
    cond = (ssort - cssv / ind) > 0
    rho_i = jnp.maximum(jnp.sum(cond, axis=1).astype(jnp.int32), 1)
    theta = jnp.take_along_axis(cssv, (rho_i - 1)[:, None], axis=1)[:, 0] / rho_i.astype(W.dtype)
    theta = jnp.maximum(theta, 0.0)
    need = a_abs.sum(axis=1) > v
    Wp = jnp.sign(W) * jnp.maximum(a_abs - theta[:, None], 0.0)
    return jnp.where(need[:, None], Wp, W)


def _stack(x_full):
    npd, m = x_full.shape
    return x_full.reshape(npd, 2, m // 2).transpose(1, 0, 2)


def kernel(features, edge_index, Ws, Os, Ps, Bs):
    src = edge_index[0]
    dst = edge_index[1]
    n = N_NODES

    pad = E_PAD - N_EDGES
    src_p = jnp.concatenate([src, jnp.zeros((pad,), jnp.int32)])
    dst_p = jnp.concatenate([dst, jnp.full((pad,), N_NODES, jnp.int32)])
    src3 = src_p.reshape(NS, NCHUNK, 1, CHUNK)
    dst3 = dst_p.reshape(NS, NCHUNK, 1, CHUNK)
    idx5 = jnp.stack([jnp.concatenate([src3, dst3], axis=2),
                      jnp.concatenate([src3 + N_PAD, dst3], axis=2)])

    vinit = jnp.full((N_PAD,), 1.0 / jnp.sqrt(jnp.float32(n)), jnp.float32)
    sa, sb = _make_spectral()(src_p.reshape(NS, SP_NCHUNK, SP_CHUNK),
                              dst_p.reshape(NS, SP_NCHUNK, SP_CHUNK), vinit)
    adj_rho = jnp.sqrt(jnp.sum(sa) / (jnp.sum(sb) / 1024.0))

    x = _stack(jnp.pad(features.T, ((0, N_PAD - n), (0, 0))))
    for i in range(N_LAYERS):
        W = _proj_linf(Ws[i], KAPPA / adj_rho)
        B = _stk_matmul(_spmm(x, idx5), Os[i].T)
        X = B
        for _ in range(FP_ITERS):
            X = jax.nn.relu(_stk_matmul(_spmm(X, idx5), W.T) + B)
        bias_stk = Bs[i].reshape(2, 1, -1)
        x = X + (_stk_matmul(x, Ps[i].T) + bias_stk)
        if i + 1 < N_LAYERS:
            x = jax.nn.elu(x)
    return jnp.concatenate([x[0, :n], x[1, :n]], axis=1)


# speedup vs baseline: 1.0037x; 1.0037x over previous
import functools

import jax
import jax.numpy as jnp
from jax import lax
from jax.experimental import pallas as pl
from jax.experimental.pallas import tpu as pltpu
from jax.experimental.pallas import tpu_sc as plsc

N_NODES = 10000
N_PAD = 10240
N_EDGES = 320000
NFEAT = 128
NHID = 32
NCLASS = 32
KAPPA = 0.9
FP_ITERS = 15
N_LAYERS = 5

NC = 2
NS = 16
CHUNK = 128
NCHUNK = 162
EPT = NCHUNK * CHUNK
E_PAD = NS * EPT
RING = 10
LAG_S = 4
LAG_I = 3
ROWS_PER_TILE = N_PAD // NS


def _spmm_body(mc, x_hbm, idx_hbm, zeros_hbm, out_hbm,
               idxr_v, rows_v, acc_sh, isem, gsem, ssem):
    cid = lax.axis_index("c")
    sid = lax.axis_index("s")

    pltpu.sync_copy(zeros_hbm.at[pl.ds(sid * ROWS_PER_TILE, ROWS_PER_TILE)],
                    acc_sh.at[pl.ds(sid * ROWS_PER_TILE, ROWS_PER_TILE)])
    plsc.subcore_barrier()

    for j in range(LAG_I):
        pltpu.async_copy(idx_hbm.at[cid, sid, j], idxr_v.at[j], isem.at[j])

    def step(k, c):
        ki = k + LAG_I

        @pl.when(ki < NCHUNK)
        def _():
            bi = lax.rem(ki, RING)
            pltpu.async_copy(idx_hbm.at[cid, sid, ki],
                             idxr_v.at[bi], isem.at[bi])

        @pl.when(k < NCHUNK)
        def _():
            b = lax.rem(k, RING)
            pltpu.make_async_copy(idx_hbm.at[cid, sid, k], idxr_v.at[b],
                                  isem.at[b]).wait()

            @pl.when(k >= RING)
            def _():
                bb = lax.rem(k - RING, RING)
                pltpu.make_async_copy(rows_v.at[bb],
                                      acc_sh.at[idxr_v.at[bb, 1]],
                                      ssem.at[bb]).wait()

            pltpu.async_copy(x_hbm.at[idxr_v.at[b, 0]], rows_v.at[b], gsem.at[b])

        @pl.when(k >= LAG_S)
        def _():
            b2 = lax.rem(k - LAG_S, RING)
            pltpu.make_async_copy(x_hbm.at[idxr_v.at[b2, 0]], rows_v.at[b2],
                                  gsem.at[b2]).wait()
            pltpu.async_copy(rows_v.at[b2], acc_sh.at[idxr_v.at[b2, 1]],
                             ssem.at[b2], add=True)
        return c

    lax.fori_loop(0, NCHUNK + LAG_S, step, 0)

    def drain(k, c):
        b = lax.rem(k, RING)
        pltpu.make_async_copy(rows_v.at[b], acc_sh.at[idxr_v.at[b, 1]],
                              ssem.at[b]).wait()
        return c
    lax.fori_loop(NCHUNK - RING, NCHUNK, drain, 0)
    plsc.subcore_barrier()

    pltpu.sync_copy(acc_sh.at[pl.ds(sid * ROWS_PER_TILE, ROWS_PER_TILE)],
                    out_hbm.at[cid, pl.ds(sid * ROWS_PER_TILE, ROWS_PER_TILE)])


@functools.lru_cache(maxsize=None)
def _make_spmm(mc):
    mesh = plsc.VectorSubcoreMesh(core_axis_name="c", subcore_axis_name="s",
                                  num_cores=NC, num_subcores=NS)
    return pl.kernel(
        functools.partial(_spmm_body, mc),
        out_type=jax.ShapeDtypeStruct((NC, N_PAD, mc), jnp.float32),
        mesh=mesh,
        compiler_params=pltpu.CompilerParams(use_tc_tiling_on_sc=False),
        scratch_types=[
            pltpu.VMEM((RING, 2, CHUNK), jnp.int32),
            pltpu.VMEM((RING, CHUNK, mc), jnp.float32),
            pltpu.VMEM_SHARED((N_PAD, mc), jnp.float32),
            pltpu.SemaphoreType.DMA((RING,)),
            pltpu.SemaphoreType.DMA((RING,)),
            pltpu.SemaphoreType.DMA((RING,)),
        ],
    )


def _spmm(x_stk, idx5):
    mc = x_stk.shape[2]
    zeros = jnp.zeros((N_PAD, mc), jnp.float32)
    return _make_spmm(mc)(x_stk.reshape(2 * N_PAD, mc), idx5, zeros)


def _stk_matmul(s_stk, Wt):
    m_in = 2 * s_stk.shape[2]
    m_out = Wt.shape[1]
    W4 = Wt.reshape(2, m_in // 2, 2, m_out // 2)
    return jnp.einsum("cnk,ckdj->dnj", s_stk, W4)


SP_CHUNK = 128
SP_NCHUNK = 162
SP_ITERS = 50


def _spectral_body(src_hbm, dst_hbm, vinit_hbm, outa_hbm, outb_hbm,
                   sidx_v, didx_v, av_v, vals_v, vsl_v, zb_v, obuf_v, sums_v,
                   v_sh, av_sh, gsem, ssem):
    cid = lax.axis_index("c")
    sid = lax.axis_index("s")
    rsl = pl.ds(sid * ROWS_PER_TILE, ROWS_PER_TILE)

    pltpu.sync_copy(src_hbm.at[sid], sidx_v)
    pltpu.sync_copy(dst_hbm.at[sid], didx_v)
    pltpu.sync_copy(vinit_hbm.at[rsl], v_sh.at[rsl])

    def z_body(i, c):
        zb_v[pl.ds(pl.multiple_of(i * 16, 16), 16)] = jnp.zeros((16,), jnp.float32)
        return c
    lax.fori_loop(0, ROWS_PER_TILE // 16, z_body, 0)
    pltpu.sync_copy(zb_v, av_sh.at[rsl])
    plsc.subcore_barrier()

    def iter_body(t, c0):
        def g_body(k, c):
            pltpu.async_copy(v_sh.at[sidx_v.at[k]], vals_v.at[k], gsem)
            return c
        lax.fori_loop(0, SP_NCHUNK, g_body, 0)

        def gw_body(k, c):
            pltpu.make_async_copy(v_sh.at[sidx_v.at[k]], vals_v.at[k],
                                  gsem).wait()
            return c
        lax.fori_loop(0, SP_NCHUNK, gw_body, 0)

        def s_body(k, c):
            pltpu.async_copy(vals_v.at[k], av_sh.at[didx_v.at[k]],
                             ssem, add=True)
            return c
        lax.fori_loop(0, SP_NCHUNK, s_body, 0)

        def w_body(k, c):
            pltpu.make_async_copy(vals_v.at[k], av_sh.at[didx_v.at[k]],
                                  ssem).wait()
            return c
        lax.fori_loop(0, SP_NCHUNK, w_body, 0)
        plsc.subcore_barrier()

        pltpu.sync_copy(av_sh, av_v)
        plsc.subcore_barrier()
        pltpu.sync_copy(zb_v, av_sh.at[rsl])

        obuf_v[...] = jnp.zeros((16,), jnp.float32)
        def r_body(i, c):
            xv = av_v[pl.ds(pl.multiple_of(i * 16, 16), 16)]
            obuf_v[...] = obuf_v[...] + xv * xv
            return c
        lax.fori_loop(0, N_NODES // 16, r_body, 0)
        sums_v[pl.ds(pl.multiple_of((t % 2) * 16, 16), 16)] = obuf_v[...]

        def v_body(i, c):
            vsl_v[pl.ds(pl.multiple_of(i * 16, 16), 16)] = (
                av_v[pl.ds(pl.multiple_of(sid * ROWS_PER_TILE + i * 16, 16), 16)]
                * jnp.float32(1.0 / 32.0))
            return c
        lax.fori_loop(0, ROWS_PER_TILE // 16, v_body, 0)
        pltpu.sync_copy(vsl_v, v_sh.at[rsl])
        plsc.subcore_barrier()
        return c0

    lax.fori_loop(0, SP_ITERS, iter_body, 0)

    @pl.when(jnp.logical_and(cid == 0, sid == 0))
    def _():
        obuf_v[...] = sums_v[pl.ds(16, 16)]
        pltpu.sync_copy(obuf_v, outa_hbm)
        obuf_v[...] = sums_v[pl.ds(0, 16)]
        pltpu.sync_copy(obuf_v, outb_hbm)


@functools.lru_cache(maxsize=None)
def _make_spectral():
    mesh = plsc.VectorSubcoreMesh(core_axis_name="c", subcore_axis_name="s",
                                  num_cores=NC, num_subcores=NS)
    return pl.kernel(
        _spectral_body,
        out_type=[jax.ShapeDtypeStruct((16,), jnp.float32),
                  jax.ShapeDtypeStruct((16,), jnp.float32)],
        mesh=mesh,
        compiler_params=pltpu.CompilerParams(use_tc_tiling_on_sc=False),
        scratch_types=[
            pltpu.VMEM((SP_NCHUNK, SP_CHUNK), jnp.int32),
            pltpu.VMEM((SP_NCHUNK, SP_CHUNK), jnp.int32),
            pltpu.VMEM((N_PAD,), jnp.float32),
            pltpu.VMEM((SP_NCHUNK, SP_CHUNK), jnp.float32),
            pltpu.VMEM((ROWS_PER_TILE,), jnp.float32),
            pltpu.VMEM((ROWS_PER_TILE,), jnp.float32),
            pltpu.VMEM((16,), jnp.float32),
            pltpu.VMEM((32,), jnp.float32),
            pltpu.VMEM_SHARED((N_PAD,), jnp.float32),
            pltpu.VMEM_SHARED((N_PAD,), jnp.float32),
            pltpu.SemaphoreType.DMA,
            pltpu.SemaphoreType.DMA,
        ],
    )


def _proj_linf(W, v):
    a_abs = jnp.abs(W)
    ssort = -jnp.sort(-a_abs, axis=1)
    cssv = jnp.cumsum(ssort, axis=1) - v
    ind = jnp.arange(1, W.shape[1] + 1, dtype=W.dtype)
    cond = (ssort - cssv / ind) > 0
    rho_i = jnp.maximum(jnp.sum(cond, axis=1).astype(jnp.int32), 1)
    theta = jnp.take_along_axis(cssv, (rho_i - 1)[:, None], axis=1)[:, 0] / rho_i.astype(W.dtype)
    theta = jnp.maximum(theta, 0.0)
    need = a_abs.sum(axis=1) > v
    Wp = jnp.sign(W) * jnp.maximum(a_abs - theta[:, None], 0.0)
    return jnp.where(need[:, None], Wp, W)


def _stack(x_full):
    npd, m = x_full.shape
    return x_full.reshape(npd, 2, m // 2).transpose(1, 0, 2)


def kernel(features, edge_index, Ws, Os, Ps, Bs):
    src = edge_index[0]
    dst = edge_index[1]
    n = N_NODES

    pad = E_PAD - N_EDGES
    src_p = jnp.concatenate([src, jnp.zeros((pad,), jnp.int32)])
    dst_p = jnp.concatenate([dst, jnp.full((pad,), N_NODES, jnp.int32)])
    src3 = src_p.reshape(NS, NCHUNK, 1, CHUNK)
    dst3 = dst_p.reshape(NS, NCHUNK, 1, CHUNK)
    idx5 = jnp.stack([jnp.concatenate([src3, dst3], axis=2),
                      jnp.concatenate([src3 + N_PAD, dst3], axis=2)])

    vinit = jnp.full((N_PAD,), 1.0 / jnp.sqrt(jnp.float32(n)), jnp.float32)
    sa, sb = _make_spectral()(src_p.reshape(NS, SP_NCHUNK, SP_CHUNK),
                              dst_p.reshape(NS, SP_NCHUNK, SP_CHUNK), vinit)
    adj_rho = jnp.sqrt(jnp.sum(sa) / (jnp.sum(sb) / 1024.0))

    x = _stack(jnp.pad(features.T, ((0, N_PAD - n), (0, 0))))
    for i in range(N_LAYERS):
        W = _proj_linf(Ws[i], KAPPA / adj_rho)
        B = _stk_matmul(_spmm(x, idx5), Os[i].T)
        X = B
        for _ in range(FP_ITERS):
            X = jax.nn.relu(_stk_matmul(_spmm(X, idx5), W.T) + B)
        bias_stk = Bs[i].reshape(2, 1, -1)
        x = X + (_stk_matmul(x, Ps[i].T) + bias_stk)
        if i + 1 < N_LAYERS:
            x = jax.nn.elu(x)
    return jnp.concatenate([x[0, :n], x[1, :n]], axis=1)


# speedup vs baseline: 1.0427x; 1.0388x over previous
import functools

import jax
import jax.numpy as jnp
from jax import lax
from jax.experimental import pallas as pl
from jax.experimental.pallas import tpu as pltpu
from jax.experimental.pallas import tpu_sc as plsc

N_NODES = 10000
N_PAD = 10240
N_EDGES = 320000
NFEAT = 128
NHID = 32
NCLASS = 32
KAPPA = 0.9
FP_ITERS = 15
N_LAYERS = 5

NC = 2
NS = 16
CHUNK = 128
NCHUNK = 162
EPT = NCHUNK * CHUNK
E_PAD = NS * EPT
RING = 10
LAG_S = 4
LAG_I = 3
ROWS_PER_TILE = N_PAD // NS


def _spmm_body(mc, x_hbm, idx_hbm, zeros_hbm, out_hbm,
               idxr_v, rows_v, acc_sh, isem, gsem, ssem):
    cid = lax.axis_index("c")
    sid = lax.axis_index("s")

    pltpu.sync_copy(zeros_hbm.at[pl.ds(sid * ROWS_PER_TILE, ROWS_PER_TILE)],
                    acc_sh.at[pl.ds(sid * ROWS_PER_TILE, ROWS_PER_TILE)])
    plsc.subcore_barrier()

    for j in range(LAG_I):
        pltpu.async_copy(idx_hbm.at[cid, sid, j], idxr_v.at[j], isem.at[j])

    def step(k, c):
        ki = k + LAG_I

        @pl.when(ki < NCHUNK)
        def _():
            bi = lax.rem(ki, RING)
            pltpu.async_copy(idx_hbm.at[cid, sid, ki],
                             idxr_v.at[bi], isem.at[bi])

        @pl.when(k < NCHUNK)
        def _():
            b = lax.rem(k, RING)
            pltpu.make_async_copy(idx_hbm.at[cid, sid, k], idxr_v.at[b],
                                  isem.at[b]).wait()

            @pl.when(k >= RING)
            def _():
                bb = lax.rem(k - RING, RING)
                pltpu.make_async_copy(rows_v.at[bb],
                                      acc_sh.at[idxr_v.at[bb, 1]],
                                      ssem.at[bb]).wait()

            pltpu.async_copy(x_hbm.at[idxr_v.at[b, 0]], rows_v.at[b], gsem.at[b])

        @pl.when(k >= LAG_S)
        def _():
            b2 = lax.rem(k - LAG_S, RING)
            pltpu.make_async_copy(x_hbm.at[idxr_v.at[b2, 0]], rows_v.at[b2],
                                  gsem.at[b2]).wait()
            pltpu.async_copy(rows_v.at[b2], acc_sh.at[idxr_v.at[b2, 1]],
                             ssem.at[b2], add=True)
        return c

    lax.fori_loop(0, NCHUNK + LAG_S, step, 0)

    def drain(k, c):
        b = lax.rem(k, RING)
        pltpu.make_async_copy(rows_v.at[b], acc_sh.at[idxr_v.at[b, 1]],
                              ssem.at[b]).wait()
        return c
    lax.fori_loop(NCHUNK - RING, NCHUNK, drain, 0)
    plsc.subcore_barrier()

    pltpu.sync_copy(acc_sh.at[pl.ds(sid * ROWS_PER_TILE, ROWS_PER_TILE)],
                    out_hbm.at[cid, pl.ds(sid * ROWS_PER_TILE, ROWS_PER_TILE)])


@functools.lru_cache(maxsize=None)
def _make_spmm(mc):
    mesh = plsc.VectorSubcoreMesh(core_axis_name="c", subcore_axis_name="s",
                                  num_cores=NC, num_subcores=NS)
    return pl.kernel(
        functools.partial(_spmm_body, mc),
        out_type=jax.ShapeDtypeStruct((NC, N_PAD, mc), jnp.float32),
        mesh=mesh,
        compiler_params=pltpu.CompilerParams(use_tc_tiling_on_sc=False),
        scratch_types=[
            pltpu.VMEM((RING, 2, CHUNK), jnp.int32),
            pltpu.VMEM((RING, CHUNK, mc), jnp.float32),
            pltpu.VMEM_SHARED((N_PAD, mc), jnp.float32),
            pltpu.SemaphoreType.DMA((RING,)),
            pltpu.SemaphoreType.DMA((RING,)),
            pltpu.SemaphoreType.DMA((RING,)),
        ],
    )


def _spmm(x_stk, idx5):
    mc = x_stk.shape[2]
    zeros = jnp.zeros((N_PAD, mc), jnp.float32)
    return _make_spmm(mc)(x_stk.reshape(2 * N_PAD, mc), idx5, zeros)


TC_BLK = 1024


def _tc_body(act, s_ref, w_ref, c_ref, b_ref, o_ref):
    sv = s_ref[...]
    wv = w_ref[...]
    cv = c_ref[...]
    bv = b_ref[...]
    outs = []
    for d in range(2):
        o = (jnp.dot(sv[0], wv[0, :, d], preferred_element_type=jnp.float32)
             + jnp.dot(sv[1], wv[1, :, d], preferred_element_type=jnp.float32)
             + cv[d] + bv[d][None, :])
        if act == 1:
            o = jnp.maximum(o, 0.0)
        elif act == 2:
            o = jnp.where(o > 0, o, jnp.exp(o) - 1.0)
        outs.append(o)
    o_ref[...] = jnp.stack(outs)


@functools.lru_cache(maxsize=None)
def _make_tc_mm(k2, m2, act):
    grid = (N_PAD // TC_BLK,)
    return pl.pallas_call(
        functools.partial(_tc_body, act),
        grid=grid,
        in_specs=[
            pl.BlockSpec((2, TC_BLK, k2), lambda i: (0, i, 0)),
            pl.BlockSpec((2, k2, 2, m2), lambda i: (0, 0, 0, 0)),
            pl.BlockSpec((2, TC_BLK, m2), lambda i: (0, i, 0)),
            pl.BlockSpec((2, m2), lambda i: (0, 0)),
        ],
        out_specs=pl.BlockSpec((2, TC_BLK, m2), lambda i: (0, i, 0)),
        out_shape=jax.ShapeDtypeStruct((2, N_PAD, m2), jnp.float32),
    )


def _stk_matmul(s_stk, Wt, C=None, bias=None, act=0):
    k2 = s_stk.shape[2]
    m_out = Wt.shape[1]
    m2 = m_out // 2
    W4 = Wt.reshape(2, k2, 2, m2)
    if C is None:
        C = jnp.zeros((2, N_PAD, m2), jnp.float32)
    if bias is None:
        bias = jnp.zeros((2, m2), jnp.float32)
    else:
        bias = bias.reshape(2, m2)
    return _make_tc_mm(k2, m2, act)(s_stk, W4, C, bias)


SP_CHUNK = 128
SP_NCHUNK = 162
SP_ITERS = 50


def _spectral_body(src_hbm, dst_hbm, vinit_hbm, outa_hbm, outb_hbm,
                   sidx_v, didx_v, av_v, vals_v, vsl_v, zb_v, obuf_v, sums_v,
                   v_sh, av_sh, gsem, ssem):
    cid = lax.axis_index("c")
    sid = lax.axis_index("s")
    rsl = pl.ds(sid * ROWS_PER_TILE, ROWS_PER_TILE)

    pltpu.sync_copy(src_hbm.at[sid], sidx_v)
    pltpu.sync_copy(dst_hbm.at[sid], didx_v)
    pltpu.sync_copy(vinit_hbm.at[rsl], v_sh.at[rsl])

    def z_body(i, c):
        zb_v[pl.ds(pl.multiple_of(i * 16, 16), 16)] = jnp.zeros((16,), jnp.float32)
        return c
    lax.fori_loop(0, ROWS_PER_TILE // 16, z_body, 0)
    pltpu.sync_copy(zb_v, av_sh.at[rsl])
    plsc.subcore_barrier()

    def iter_body(t, c0):
        def g_body(k, c):
            pltpu.async_copy(v_sh.at[sidx_v.at[k]], vals_v.at[k], gsem)
            return c
        lax.fori_loop(0, SP_NCHUNK, g_body, 0)

        def gw_body(k, c):
            pltpu.make_async_copy(v_sh.at[sidx_v.at[k]], vals_v.at[k],
                                  gsem).wait()
            return c
        lax.fori_loop(0, SP_NCHUNK, gw_body, 0)

        def s_body(k, c):
            pltpu.async_copy(vals_v.at[k], av_sh.at[didx_v.at[k]],
                             ssem, add=True)
            return c
        lax.fori_loop(0, SP_NCHUNK, s_body, 0)

        def w_body(k, c):
            pltpu.make_async_copy(vals_v.at[k], av_sh.at[didx_v.at[k]],
                                  ssem).wait()
            return c
        lax.fori_loop(0, SP_NCHUNK, w_body, 0)
        plsc.subcore_barrier()

        pltpu.sync_copy(av_sh, av_v)
        plsc.subcore_barrier()
        pltpu.sync_copy(zb_v, av_sh.at[rsl])

        obuf_v[...] = jnp.zeros((16,), jnp.float32)
        def r_body(i, c):
            xv = av_v[pl.ds(pl.multiple_of(i * 16, 16), 16)]
            obuf_v[...] = obuf_v[...] + xv * xv
            return c
        lax.fori_loop(0, N_NODES // 16, r_body, 0)
        sums_v[pl.ds(pl.multiple_of((t % 2) * 16, 16), 16)] = obuf_v[...]

        def v_body(i, c):
            vsl_v[pl.ds(pl.multiple_of(i * 16, 16), 16)] = (
                av_v[pl.ds(pl.multiple_of(sid * ROWS_PER_TILE + i * 16, 16), 16)]
                * jnp.float32(1.0 / 32.0))
            return c
        lax.fori_loop(0, ROWS_PER_TILE // 16, v_body, 0)
        pltpu.sync_copy(vsl_v, v_sh.at[rsl])
        plsc.subcore_barrier()
        return c0

    lax.fori_loop(0, SP_ITERS, iter_body, 0)

    @pl.when(jnp.logical_and(cid == 0, sid == 0))
    def _():
        obuf_v[...] = sums_v[pl.ds(16, 16)]
        pltpu.sync_copy(obuf_v, outa_hbm)
        obuf_v[...] = sums_v[pl.ds(0, 16)]
        pltpu.sync_copy(obuf_v, outb_hbm)


@functools.lru_cache(maxsize=None)
def _make_spectral():
    mesh = plsc.VectorSubcoreMesh(core_axis_name="c", subcore_axis_name="s",
                                  num_cores=NC, num_subcores=NS)
    return pl.kernel(
        _spectral_body,
        out_type=[jax.ShapeDtypeStruct((16,), jnp.float32),
                  jax.ShapeDtypeStruct((16,), jnp.float32)],
        mesh=mesh,
        compiler_params=pltpu.CompilerParams(use_tc_tiling_on_sc=False),
        scratch_types=[
            pltpu.VMEM((SP_NCHUNK, SP_CHUNK), jnp.int32),
            pltpu.VMEM((SP_NCHUNK, SP_CHUNK), jnp.int32),
            pltpu.VMEM((N_PAD,), jnp.float32),
            pltpu.VMEM((SP_NCHUNK, SP_CHUNK), jnp.float32),
            pltpu.VMEM((ROWS_PER_TILE,), jnp.float32),
            pltpu.VMEM((ROWS_PER_TILE,), jnp.float32),
            pltpu.VMEM((16,), jnp.float32),
            pltpu.VMEM((32,), jnp.float32),
            pltpu.VMEM_SHARED((N_PAD,), jnp.float32),
            pltpu.VMEM_SHARED((N_PAD,), jnp.float32),
            pltpu.SemaphoreType.DMA,
            pltpu.SemaphoreType.DMA,
        ],
    )


def _proj_linf(W, v):
    a_abs = jnp.abs(W)
    ssort = -jnp.sort(-a_abs, axis=1)
    cssv = jnp.cumsum(ssort, axis=1) - v
    ind = jnp.arange(1, W.shape[1] + 1, dtype=W.dtype)
    cond = (ssort - cssv / ind) > 0
    rho_i = jnp.maximum(jnp.sum(cond, axis=1).astype(jnp.int32), 1)
    theta = jnp.take_along_axis(cssv, (rho_i - 1)[:, None], axis=1)[:, 0] / rho_i.astype(W.dtype)
    theta = jnp.maximum(theta, 0.0)
    need = a_abs.sum(axis=1) > v
    Wp = jnp.sign(W) * jnp.maximum(a_abs - theta[:, None], 0.0)
    return jnp.where(need[:, None], Wp, W)


def _stack(x_full):
    npd, m = x_full.shape
    return x_full.reshape(npd, 2, m // 2).transpose(1, 0, 2)


def kernel(features, edge_index, Ws, Os, Ps, Bs):
    src = edge_index[0]
    dst = edge_index[1]
    n = N_NODES

    pad = E_PAD - N_EDGES
    src_p = jnp.concatenate([src, jnp.zeros((pad,), jnp.int32)])
    dst_p = jnp.concatenate([dst, jnp.full((pad,), N_NODES, jnp.int32)])
    src3 = src_p.reshape(NS, NCHUNK, 1, CHUNK)
    dst3 = dst_p.reshape(NS, NCHUNK, 1, CHUNK)
    idx5 = jnp.stack([jnp.concatenate([src3, dst3], axis=2),
                      jnp.concatenate([src3 + N_PAD, dst3], axis=2)])

    vinit = jnp.full((N_PAD,), 1.0 / jnp.sqrt(jnp.float32(n)), jnp.float32)
    sa, sb = _make_spectral()(src_p.reshape(NS, SP_NCHUNK, SP_CHUNK),
                              dst_p.reshape(NS, SP_NCHUNK, SP_CHUNK), vinit)
    adj_rho = jnp.sqrt(jnp.sum(sa) / (jnp.sum(sb) / 1024.0))

    x = _stack(jnp.pad(features.T, ((0, N_PAD - n), (0, 0))))
    for i in range(N_LAYERS):
        W = _proj_linf(Ws[i], KAPPA / adj_rho)
        B = _stk_matmul(_spmm(x, idx5), Os[i].T)
        X = B
        for _ in range(FP_ITERS):
            X = _stk_matmul(_spmm(X, idx5), W.T, C=B, act=1)
        x = _stk_matmul(x, Ps[i].T, C=X, bias=Bs[i],
                        act=2 if i + 1 < N_LAYERS else 0)
    return jnp.concatenate([x[0, :n], x[1, :n]], axis=1)
